# Initial kernel scaffold; baseline (speedup 1.0000x reference)
import jax
import jax.numpy as jnp
from jax.experimental import pallas as pl


def kernel(x, edge_index, edge_attr, batch, node_table, edge_table, W1, b1, W2, b2, eps0, Wa, Wb, eps_mp, L1a, L1b, L2a, L2b, Wf):
    raise NotImplementedError("write your pallas kernel here")


# speedup vs baseline: 2.9943x; 2.9943x over previous
import functools
import math

import jax
import jax.numpy as jnp
from jax import lax
from jax.experimental import pallas as pl
from jax.experimental.pallas import tpu as pltpu
from jax.experimental.pallas import tpu_sc as plsc

N = 10000
E = 320000
H = 128
NG = 64
NEA = 5

CBN = 1.0 / math.sqrt(1.0 + 1e-5)

NC = 2
NS = 16
W = NC * NS
CH = 64
NCH = 160
EPAD = W * NCH * CH
GRP = 2
NPH = 2
NACC = 10240
XPW = 320


@functools.cache
def _embed_sc_build():
    mesh = plsc.VectorSubcoreMesh(core_axis_name="c", subcore_axis_name="s")
    return functools.partial(
        pl.kernel,
        out_type=jax.ShapeDtypeStruct((W * XPW, H), jnp.float32),
        mesh=mesh,
        scratch_types=[
            pltpu.VMEM((XPW // 64, 64), jnp.int32),
            pltpu.VMEM((64, H), jnp.float32),
            pltpu.SemaphoreType.DMA,
        ],
    )(_embed_sc_body)


def _embed_sc(table, x_p):
    return _embed_sc_build()(table, x_p)


def _embed_sc_body(table_hbm, xr_hbm, out_hbm, xv, rows, sem):
    c = lax.axis_index("c")
    s = lax.axis_index("s")
    w = c * NS + s
    pltpu.sync_copy(xr_hbm.at[w], xv)
    for j in range(XPW // 64):
        pltpu.async_copy(table_hbm.at[xv.at[j]], rows, sem).wait()
        pltpu.sync_copy(rows, out_hbm.at[pl.ds(w * XPW + j * 64, 64)])


@functools.cache
def _agg_sc_build():
    mesh = plsc.VectorSubcoreMesh(core_axis_name="c", subcore_axis_name="s")
    return functools.partial(
        pl.kernel,
        out_type=jax.ShapeDtypeStruct((NC, NACC, H), jnp.float32),
        mesh=mesh,
        scratch_types=[
            pltpu.VMEM((NCH // NPH, CH), jnp.int32),
            pltpu.VMEM((NCH // NPH, CH), jnp.int32),
            pltpu.VMEM((CH, H), jnp.float32),
            pltpu.VMEM((CH, H), jnp.float32),
            pltpu.VMEM_SHARED((NACC, H), jnp.float32),
            pltpu.SemaphoreType.DMA,
        ],
    )(_agg_sc_body)


def _agg_sc(haug, cidx_p, dst_p):
    return _agg_sc_build()(haug, cidx_p, dst_p)


def _agg_sc_body(haug_hbm, cidx_hbm, dst_hbm, out_hbm,
                 cidx_v, dst_v, r0, r1, acc, sem):
    c = lax.axis_index("c")
    s = lax.axis_index("s")
    w = c * NS + s
    rows = [r0, r1]
    pch = NCH // NPH

    def _zrow(i, carry):
        for g in range(H // 16):
            r0[i, pl.ds(g * 16, 16)] = jnp.zeros((16,), jnp.float32)
        return carry

    lax.fori_loop(0, CH, _zrow, 0)
    spw = NACC // NS
    for j in range(spw // CH):
        pltpu.sync_copy(r0, acc.at[pl.ds(s * spw + j * CH, CH)])
    plsc.subcore_barrier()

    def _group(g, carry):
        base = g * GRP
        cps = [
            pltpu.async_copy(haug_hbm.at[cidx_v.at[base + b]], rows[b], sem)
            for b in range(GRP)
        ]
        for b in range(GRP):
            cps[b].wait()
        for b in range(GRP):
            pltpu.sync_copy(rows[b], acc.at[dst_v.at[base + b]], add=True)
        return carry

    for ph in range(NPH):
        pltpu.sync_copy(cidx_hbm.at[w, pl.ds(ph * pch, pch)], cidx_v)
        pltpu.sync_copy(dst_hbm.at[w, pl.ds(ph * pch, pch)], dst_v)
        lax.fori_loop(0, pch // GRP, _group, 0)
    plsc.subcore_barrier()
    pltpu.sync_copy(acc.at[pl.ds(s * spw, spw)], out_hbm.at[c, pl.ds(s * spw, spw)])


def _haug_body(h_ref, t_ref, out_ref):
    h = h_ref[...]
    for a in range(NEA):
        out_ref[a] = jnp.maximum(h + t_ref[a], 0.0)


def _haug_tc(h, tpad):
    bn = 1000
    out = pl.pallas_call(
        _haug_body,
        grid=(N // bn,),
        in_specs=[
            pl.BlockSpec((bn, H), lambda i: (i, 0)),
            pl.BlockSpec((8, H), lambda i: (0, 0)),
        ],
        out_specs=pl.BlockSpec((NEA, bn, H), lambda i: (0, i, 0)),
        out_shape=jax.ShapeDtypeStruct((NEA, N, H), jnp.float32),
    )(h, tpad)
    return out


def _mlp_body(emit_haug, hs_ref, h_ref, p0_ref, p1_ref, w1_ref, b1_ref,
              w2_ref, b2_ref, t_ref, *out_refs):
    hs = hs_ref[0, 0]
    z = h_ref[...] * hs + p0_ref[...] + p1_ref[...]
    y = jnp.dot(z, w1_ref[...], preferred_element_type=jnp.float32)
    y = jnp.maximum(y * CBN + b1_ref[...] * CBN, 0.0)
    hn = jnp.dot(y, w2_ref[...], preferred_element_type=jnp.float32)
    hn = jnp.maximum(hn * CBN + b2_ref[...] * CBN, 0.0)
    out_refs[0][...] = hn
    if emit_haug:
        for a in range(NEA):
            out_refs[1][a] = jnp.maximum(hn + t_ref[a], 0.0)


def _mlp_tc(h, p0, p1, w1, b1, w2, b2, hscale, tpad, emit_haug):
    bn = 1000
    out_shapes = [jax.ShapeDtypeStruct((N, H), jnp.float32)]
    out_specs = [pl.BlockSpec((bn, H), lambda i: (i, 0))]
    if emit_haug:
        out_shapes.append(jax.ShapeDtypeStruct((NEA, N, H), jnp.float32))
        out_specs.append(pl.BlockSpec((NEA, bn, H), lambda i: (0, i, 0)))
    outs = pl.pallas_call(
        functools.partial(_mlp_body, emit_haug),
        grid=(N // bn,),
        in_specs=[
            pl.BlockSpec(memory_space=pltpu.SMEM),
            pl.BlockSpec((bn, H), lambda i: (i, 0)),
            pl.BlockSpec((bn, H), lambda i: (i, 0)),
            pl.BlockSpec((bn, H), lambda i: (i, 0)),
            pl.BlockSpec((H, H), lambda i: (0, 0)),
            pl.BlockSpec((1, H), lambda i: (0, 0)),
            pl.BlockSpec((H, H), lambda i: (0, 0)),
            pl.BlockSpec((1, H), lambda i: (0, 0)),
            pl.BlockSpec((8, H), lambda i: (0, 0)),
        ],
        out_specs=out_specs,
        out_shape=out_shapes,
    )(hscale, h, p0, p1, w1, b1, w2, b2, tpad)
    return outs


def _readout_body(batch_ref, h_ref, l1a_ref, l1b_ref, l2a_ref, l2b_ref,
                  wf_ref, out_ref, acc_ref):
    i = pl.program_id(0)
    bn = h_ref.shape[0]

    @pl.when(i == 0)
    def _():
        acc_ref[...] = jnp.zeros_like(acc_ref)

    b = batch_ref[0]
    gid = lax.broadcasted_iota(jnp.int32, (NG, bn), 0)
    onehot = (gid == b).astype(jnp.float32)
    acc_ref[...] += jnp.dot(onehot, h_ref[...], preferred_element_type=jnp.float32)

    @pl.when(i == pl.num_programs(0) - 1)
    def _():
        g = acc_ref[...]
        g = jnp.maximum(jnp.dot(g, l1a_ref[...], preferred_element_type=jnp.float32) * CBN, 0.0)
        g = jnp.maximum(jnp.dot(g, l1b_ref[...], preferred_element_type=jnp.float32) * CBN, 0.0)
        g = jnp.maximum(jnp.dot(g, l2a_ref[...], preferred_element_type=jnp.float32) * CBN, 0.0)
        g = jnp.maximum(jnp.dot(g, l2b_ref[...], preferred_element_type=jnp.float32) * CBN, 0.0)
        out_ref[...] = jnp.dot(g, wf_ref[...], preferred_element_type=jnp.float32)


def _readout_tc(batch_r, h, l1a, l1b, l2a, l2b, wfpad):
    bn = 1000
    out = pl.pallas_call(
        _readout_body,
        grid=(N // bn,),
        in_specs=[
            pl.BlockSpec((1, 1, bn), lambda i: (i, 0, 0)),
            pl.BlockSpec((bn, H), lambda i: (i, 0)),
            pl.BlockSpec((H, H), lambda i: (0, 0)),
            pl.BlockSpec((H, H), lambda i: (0, 0)),
            pl.BlockSpec((H, H), lambda i: (0, 0)),
            pl.BlockSpec((H, H), lambda i: (0, 0)),
            pl.BlockSpec((H, H), lambda i: (0, 0)),
        ],
        out_specs=pl.BlockSpec((NG, H), lambda i: (0, 0)),
        out_shape=jax.ShapeDtypeStruct((NG, H), jnp.float32),
        scratch_shapes=[pltpu.VMEM((NG, H), jnp.float32)],
    )(batch_r, h, l1a, l1b, l2a, l2b, wfpad)
    return out


def kernel(x, edge_index, edge_attr, batch, node_table, edge_table,
           W1, b1, W2, b2, eps0, Wa, Wb, eps_mp, L1a, L1b, L2a, L2b, Wf):
    f32 = jnp.float32
    src = edge_index[0].astype(jnp.int32)
    dst = edge_index[1].astype(jnp.int32)

    cidx = edge_attr.astype(jnp.int32) * N + src
    pad = EPAD - E
    cidx_p = jnp.concatenate([cidx, jnp.zeros((pad,), jnp.int32)]).reshape(W, NCH, CH)
    dst_p = jnp.concatenate([dst, jnp.full((pad,), N, jnp.int32)]).reshape(W, NCH, CH)
    x_p = jnp.concatenate(
        [x.astype(jnp.int32), jnp.zeros((W * XPW - N,), jnp.int32)]
    ).reshape(W, XPW // 64, 64)
    tpad = jnp.zeros((8, H), f32).at[:NEA].set(edge_table)
    wfpad = jnp.zeros((H, H), f32).at[:, :1].set(Wf)
    batch_r = batch.astype(jnp.int32).reshape(10, 1, N // 10)
    zb = jnp.zeros((1, H), f32)

    h = _embed_sc(node_table.astype(f32), x_p)[:N]
    haug = _haug_tc(h, tpad)

    for layer in range(5):
        parts = _agg_sc(haug.reshape(NEA * N, H), cidx_p, dst_p)[:, :N]
        if layer == 0:
            w1, bb1, w2, bb2 = W1, b1.reshape(1, H), W2, b2.reshape(1, H)
            hscale = (1.0 + eps0).astype(f32).reshape(1, 1)
        else:
            w1, bb1, w2, bb2 = Wa[layer - 1], zb, Wb[layer - 1], zb
            hscale = (1.0 + eps_mp[layer - 1]).astype(f32).reshape(1, 1)
        outs = _mlp_tc(h, parts[0], parts[1], w1, bb1, w2, bb2,
                       hscale, tpad, emit_haug=(layer < 4))
        h = outs[0]
        if layer < 4:
            haug = outs[1]

    g = _readout_tc(batch_r, h, L1a, L1b, L2a, L2b, wfpad)
    return g[:, :1]


# speedup vs baseline: 3.7485x; 1.2519x over previous
import functools
import math

import jax
import jax.numpy as jnp
from jax import lax
from jax.experimental import pallas as pl
from jax.experimental.pallas import tpu as pltpu
from jax.experimental.pallas import tpu_sc as plsc

N = 10000
E = 320000
H = 128
NG = 64
NEA = 5

CBN = 1.0 / math.sqrt(1.0 + 1e-5)

NC = 2
NS = 16
W = NC * NS
CH = 64
PCH = 40
SPLIT = (6, 2)
NCH0 = SPLIT[0] * PCH
NCH1 = SPLIT[1] * PCH
EPAD = NS * (NCH0 + NCH1) * CH
GRP = 2
NACC = 10240
XPW = 320


@functools.cache
def _embed_sc_build():
    mesh = plsc.VectorSubcoreMesh(core_axis_name="c", subcore_axis_name="s")
    return functools.partial(
        pl.kernel,
        out_type=jax.ShapeDtypeStruct((W * XPW, H), jnp.float32),
        mesh=mesh,
        scratch_types=[
            pltpu.VMEM((XPW // 64, 64), jnp.int32),
            pltpu.VMEM((64, H), jnp.float32),
            pltpu.SemaphoreType.DMA,
        ],
    )(_embed_sc_body)


def _embed_sc(table, x_p):
    return _embed_sc_build()(table, x_p)


def _embed_sc_body(table_hbm, xr_hbm, out_hbm, xv, rows, sem):
    c = lax.axis_index("c")
    s = lax.axis_index("s")
    w = c * NS + s
    pltpu.sync_copy(xr_hbm.at[w], xv)
    for j in range(XPW // 64):
        pltpu.async_copy(table_hbm.at[xv.at[j]], rows, sem).wait()
        pltpu.sync_copy(rows, out_hbm.at[pl.ds(w * XPW + j * 64, 64)])


@functools.cache
def _agg_sc_build():
    mesh = plsc.VectorSubcoreMesh(core_axis_name="c", subcore_axis_name="s")
    return functools.partial(
        pl.kernel,
        out_type=jax.ShapeDtypeStruct((NC, NACC, H), jnp.float32),
        mesh=mesh,
        scratch_types=[
            pltpu.VMEM((PCH, CH), jnp.int32),
            pltpu.VMEM((PCH, CH), jnp.int32),
            pltpu.VMEM((CH, H), jnp.float32),
            pltpu.VMEM((CH, H), jnp.float32),
            pltpu.VMEM_SHARED((NACC, H), jnp.float32),
            pltpu.SemaphoreType.DMA,
        ],
    )(_agg_sc_body)


def _agg_sc(haug, cidx_p, dst_p):
    return _agg_sc_build()(haug, cidx_p, dst_p)


def _agg_sc_body(haug_hbm, cidx_hbm, dst_hbm, out_hbm,
                 cidx_v, dst_v, r0, r1, acc, sem):
    c = lax.axis_index("c")
    s = lax.axis_index("s")
    w = c * NS + s
    rows = [r0, r1]
    nph = jnp.where(c == 0, SPLIT[0], SPLIT[1])

    def _zrow(i, carry):
        for g in range(H // 16):
            r0[i, pl.ds(g * 16, 16)] = jnp.zeros((16,), jnp.float32)
        return carry

    lax.fori_loop(0, CH, _zrow, 0)
    spw = NACC // NS
    for j in range(spw // CH):
        pltpu.sync_copy(r0, acc.at[pl.ds(s * spw + j * CH, CH)])
    plsc.subcore_barrier()

    def _group(g, carry):
        base = g * GRP
        cps = [
            pltpu.async_copy(haug_hbm.at[cidx_v.at[base + b]], rows[b], sem)
            for b in range(GRP)
        ]
        for b in range(GRP):
            cps[b].wait()
        for b in range(GRP):
            pltpu.sync_copy(rows[b], acc.at[dst_v.at[base + b]], add=True)
        return carry

    for ph in range(max(SPLIT)):
        @pl.when(ph < nph)
        def _():
            pltpu.sync_copy(cidx_hbm.at[w, pl.ds(ph * PCH, PCH)], cidx_v)
            pltpu.sync_copy(dst_hbm.at[w, pl.ds(ph * PCH, PCH)], dst_v)
            lax.fori_loop(0, PCH // GRP, _group, 0)
    plsc.subcore_barrier()
    pltpu.sync_copy(acc.at[pl.ds(s * spw, spw)], out_hbm.at[c, pl.ds(s * spw, spw)])


def _haug_body(h_ref, t_ref, out_ref):
    h = h_ref[...]
    for a in range(NEA):
        out_ref[a] = jnp.maximum(h + t_ref[a], 0.0)


def _haug_tc(h, tpad):
    bn = 1000
    out = pl.pallas_call(
        _haug_body,
        grid=(N // bn,),
        in_specs=[
            pl.BlockSpec((bn, H), lambda i: (i, 0)),
            pl.BlockSpec((8, H), lambda i: (0, 0)),
        ],
        out_specs=pl.BlockSpec((NEA, bn, H), lambda i: (0, i, 0)),
        out_shape=jax.ShapeDtypeStruct((NEA, N, H), jnp.float32),
    )(h, tpad)
    return out


def _mlp_body(emit_haug, hs_ref, h_ref, p0_ref, p1_ref, w1_ref, b1_ref,
              w2_ref, b2_ref, t_ref, *out_refs):
    hs = hs_ref[0, 0]
    z = h_ref[...] * hs + p0_ref[...] + p1_ref[...]
    y = jnp.dot(z, w1_ref[...], preferred_element_type=jnp.float32)
    y = jnp.maximum(y * CBN + b1_ref[...] * CBN, 0.0)
    hn = jnp.dot(y, w2_ref[...], preferred_element_type=jnp.float32)
    hn = jnp.maximum(hn * CBN + b2_ref[...] * CBN, 0.0)
    out_refs[0][...] = hn
    if emit_haug:
        for a in range(NEA):
            out_refs[1][a] = jnp.maximum(hn + t_ref[a], 0.0)


def _mlp_tc(h, p0, p1, w1, b1, w2, b2, hscale, tpad, emit_haug):
    bn = 1000
    out_shapes = [jax.ShapeDtypeStruct((N, H), jnp.float32)]
    out_specs = [pl.BlockSpec((bn, H), lambda i: (i, 0))]
    if emit_haug:
        out_shapes.append(jax.ShapeDtypeStruct((NEA, N, H), jnp.float32))
        out_specs.append(pl.BlockSpec((NEA, bn, H), lambda i: (0, i, 0)))
    outs = pl.pallas_call(
        functools.partial(_mlp_body, emit_haug),
        grid=(N // bn,),
        in_specs=[
            pl.BlockSpec(memory_space=pltpu.SMEM),
            pl.BlockSpec((bn, H), lambda i: (i, 0)),
            pl.BlockSpec((bn, H), lambda i: (i, 0)),
            pl.BlockSpec((bn, H), lambda i: (i, 0)),
            pl.BlockSpec((H, H), lambda i: (0, 0)),
            pl.BlockSpec((1, H), lambda i: (0, 0)),
            pl.BlockSpec((H, H), lambda i: (0, 0)),
            pl.BlockSpec((1, H), lambda i: (0, 0)),
            pl.BlockSpec((8, H), lambda i: (0, 0)),
        ],
        out_specs=out_specs,
        out_shape=out_shapes,
    )(hscale, h, p0, p1, w1, b1, w2, b2, tpad)
    return outs


def _readout_body(batch_ref, h_ref, l1a_ref, l1b_ref, l2a_ref, l2b_ref,
                  wf_ref, out_ref, acc_ref):
    i = pl.program_id(0)
    bn = h_ref.shape[0]

    @pl.when(i == 0)
    def _():
        acc_ref[...] = jnp.zeros_like(acc_ref)

    b = batch_ref[0]
    gid = lax.broadcasted_iota(jnp.int32, (NG, bn), 0)
    onehot = (gid == b).astype(jnp.float32)
    acc_ref[...] += jnp.dot(onehot, h_ref[...], preferred_element_type=jnp.float32)

    @pl.when(i == pl.num_programs(0) - 1)
    def _():
        g = acc_ref[...]
        g = jnp.maximum(jnp.dot(g, l1a_ref[...], preferred_element_type=jnp.float32) * CBN, 0.0)
        g = jnp.maximum(jnp.dot(g, l1b_ref[...], preferred_element_type=jnp.float32) * CBN, 0.0)
        g = jnp.maximum(jnp.dot(g, l2a_ref[...], preferred_element_type=jnp.float32) * CBN, 0.0)
        g = jnp.maximum(jnp.dot(g, l2b_ref[...], preferred_element_type=jnp.float32) * CBN, 0.0)
        out_ref[...] = jnp.dot(g, wf_ref[...], preferred_element_type=jnp.float32)


def _readout_tc(batch_r, h, l1a, l1b, l2a, l2b, wfpad):
    bn = 1000
    out = pl.pallas_call(
        _readout_body,
        grid=(N // bn,),
        in_specs=[
            pl.BlockSpec((1, 1, bn), lambda i: (i, 0, 0)),
            pl.BlockSpec((bn, H), lambda i: (i, 0)),
            pl.BlockSpec((H, H), lambda i: (0, 0)),
            pl.BlockSpec((H, H), lambda i: (0, 0)),
            pl.BlockSpec((H, H), lambda i: (0, 0)),
            pl.BlockSpec((H, H), lambda i: (0, 0)),
            pl.BlockSpec((H, H), lambda i: (0, 0)),
        ],
        out_specs=pl.BlockSpec((NG, H), lambda i: (0, 0)),
        out_shape=jax.ShapeDtypeStruct((NG, H), jnp.float32),
        scratch_shapes=[pltpu.VMEM((NG, H), jnp.float32)],
    )(batch_r, h, l1a, l1b, l2a, l2b, wfpad)
    return out


def kernel(x, edge_index, edge_attr, batch, node_table, edge_table,
           W1, b1, W2, b2, eps0, Wa, Wb, eps_mp, L1a, L1b, L2a, L2b, Wf):
    f32 = jnp.float32
    src = edge_index[0].astype(jnp.int32)
    dst = edge_index[1].astype(jnp.int32)

    cidx = edge_attr.astype(jnp.int32) * N + src
    pad = EPAD - E
    ncm = max(NCH0, NCH1)
    e0 = NS * NCH0 * CH

    def _pack(arr):
        a0 = arr[:e0].reshape(NS, NCH0, CH)
        a1 = arr[e0:].reshape(NS, NCH1, CH)
        a0 = jnp.pad(a0, ((0, 0), (0, ncm - NCH0), (0, 0)))
        a1 = jnp.pad(a1, ((0, 0), (0, ncm - NCH1), (0, 0)))
        return jnp.concatenate([a0, a1])

    cidx_p = _pack(jnp.concatenate([cidx, jnp.zeros((pad,), jnp.int32)]))
    dst_p = _pack(jnp.concatenate([dst, jnp.full((pad,), N, jnp.int32)]))
    x_p = jnp.concatenate(
        [x.astype(jnp.int32), jnp.zeros((W * XPW - N,), jnp.int32)]
    ).reshape(W, XPW // 64, 64)
    tpad = jnp.zeros((8, H), f32).at[:NEA].set(edge_table)
    wfpad = jnp.zeros((H, H), f32).at[:, :1].set(Wf)
    batch_r = batch.astype(jnp.int32).reshape(10, 1, N // 10)
    zb = jnp.zeros((1, H), f32)

    h = _embed_sc(node_table.astype(f32), x_p)[:N]
    haug = _haug_tc(h, tpad)

    for layer in range(5):
        parts = _agg_sc(haug.reshape(NEA * N, H), cidx_p, dst_p)[:, :N]
        if layer == 0:
            w1, bb1, w2, bb2 = W1, b1.reshape(1, H), W2, b2.reshape(1, H)
            hscale = (1.0 + eps0).astype(f32).reshape(1, 1)
        else:
            w1, bb1, w2, bb2 = Wa[layer - 1], zb, Wb[layer - 1], zb
            hscale = (1.0 + eps_mp[layer - 1]).astype(f32).reshape(1, 1)
        outs = _mlp_tc(h, parts[0], parts[1], w1, bb1, w2, bb2,
                       hscale, tpad, emit_haug=(layer < 4))
        h = outs[0]
        if layer < 4:
            haug = outs[1]

    g = _readout_tc(batch_r, h, L1a, L1b, L2a, L2b, wfpad)
    return g[:, :1]
